# baseline (speedup 1.0000x reference)
import jax
import jax.numpy as jnp
from jax.experimental import pallas as pl
from jax.experimental.pallas import tpu as pltpu

_BLOCK_TOKENS = 1640


def _row_scale_kernel(x_ref, g_ref, o_ref):
    s = jnp.sum(g_ref[...], axis=1, keepdims=True)
    o_ref[...] = x_ref[...] * s


def kernel(x, gates):
    tokens, d_model = x.shape
    num_experts = gates.shape[1]
    bt = _BLOCK_TOKENS
    grid = (pl.cdiv(tokens, bt),)
    return pl.pallas_call(
        _row_scale_kernel,
        grid=grid,
        in_specs=[
            pl.BlockSpec((bt, d_model), lambda i: (i, 0)),
            pl.BlockSpec((bt, num_experts), lambda i: (i, 0)),
        ],
        out_specs=pl.BlockSpec((bt, d_model), lambda i: (i, 0)),
        compiler_params=pltpu.CompilerParams(vmem_limit_bytes=67_000_000),
        out_shape=jax.ShapeDtypeStruct((tokens, d_model), x.dtype),
    )(x, gates)


# speedup vs baseline: 1.0328x; 1.0328x over previous
import jax
import jax.numpy as jnp
from jax.experimental import pallas as pl
from jax.experimental.pallas import tpu as pltpu

_BLOCK_TOKENS = 1856


def _row_scale_kernel(x_ref, g_ref, o_ref):
    s = jnp.sum(g_ref[...], axis=1, keepdims=True)
    o_ref[...] = x_ref[...] * s


def kernel(x, gates):
    tokens, d_model = x.shape
    num_experts = gates.shape[1]
    bt = _BLOCK_TOKENS
    grid = (pl.cdiv(tokens, bt),)
    return pl.pallas_call(
        _row_scale_kernel,
        grid=grid,
        in_specs=[
            pl.BlockSpec((bt, d_model), lambda i: (i, 0)),
            pl.BlockSpec((bt, num_experts), lambda i: (i, 0)),
        ],
        out_specs=pl.BlockSpec((bt, d_model), lambda i: (i, 0)),
        compiler_params=pltpu.CompilerParams(vmem_limit_bytes=67_000_000),
        out_shape=jax.ShapeDtypeStruct((tokens, d_model), x.dtype),
    )(x, gates)
